# baseline (speedup 1.0000x reference)
import dataclasses
import functools

import jax
import jax.numpy as jnp
from jax import lax
from jax.experimental import pallas as pl
from jax.experimental.pallas import tpu as pltpu
from jax.experimental.pallas import tpu_sc as plsc

_BATCH = 16384
_DIM = 32
_LANES = 16
_NUM_CORES = 2
_NUM_SUBCORES = 16
_NUM_WORKERS = _NUM_CORES * _NUM_SUBCORES
_PER_WORKER = _BATCH // _NUM_WORKERS
_CHUNK = 128
_NUM_CHUNKS = _PER_WORKER // _CHUNK


def _make_kernel(num_users, num_movies):
    mesh = plsc.VectorSubcoreMesh(core_axis_name="c", subcore_axis_name="s")
    cp = pltpu.CompilerParams()
    if "needs_layout_passes" in pltpu.CompilerParams.__dataclass_fields__:
        cp = dataclasses.replace(cp, needs_layout_passes=False)
    if "use_tc_tiling_on_sc" in pltpu.CompilerParams.__dataclass_fields__:
        cp = dataclasses.replace(cp, use_tc_tiling_on_sc=False)

    @functools.partial(
        pl.kernel,
        mesh=mesh,
        compiler_params=cp,
        out_type=jax.ShapeDtypeStruct((_BATCH,), jnp.float32),
        scratch_types=[
            pltpu.VMEM((_NUM_CHUNKS, _CHUNK), jnp.int32),
            pltpu.VMEM((_NUM_CHUNKS, _CHUNK), jnp.int32),
            pltpu.VMEM((_PER_WORKER, _DIM), jnp.float32),
            pltpu.VMEM((_PER_WORKER, _DIM), jnp.float32),
            pltpu.VMEM((_PER_WORKER,), jnp.float32),
            pltpu.VMEM((_PER_WORKER,), jnp.float32),
            pltpu.VMEM((_PER_WORKER,), jnp.float32),
            pltpu.SemaphoreType.DMA,
        ],
    )
    def k(uid_hbm, mid_hbm, uemb_hbm, memb_hbm, ubias_hbm, mbias_hbm,
          out_hbm, uidx, midx, urows, mrows, ubv, mbv, outv, sem):
        wid = lax.axis_index("s") * _NUM_CORES + lax.axis_index("c")
        base = wid * _PER_WORKER

        pltpu.sync_copy(uid_hbm.at[wid], uidx)
        pltpu.sync_copy(mid_hbm.at[wid], midx)

        copies = []
        for j in range(_NUM_CHUNKS):
            sl = pl.ds(j * _CHUNK, _CHUNK)
            copies.append(
                pltpu.async_copy(uemb_hbm.at[uidx.at[j]], urows.at[sl], sem))
            copies.append(
                pltpu.async_copy(memb_hbm.at[midx.at[j]], mrows.at[sl], sem))
            copies.append(
                pltpu.async_copy(ubias_hbm.at[uidx.at[j]], ubv.at[sl], sem))
            copies.append(
                pltpu.async_copy(mbias_hbm.at[midx.at[j]], mbv.at[sl], sem))
        for c in copies:
            c.wait()

        lanes = lax.iota(jnp.int32, _LANES)

        @pl.loop(0, _PER_WORKER // _LANES)
        def _(g):
            gbase = g * _LANES
            acc = jnp.zeros((_LANES,), jnp.float32)
            for j in range(_LANES):
                b = gbase + j
                s = (urows[b, 0:_LANES] * mrows[b, 0:_LANES]
                     + urows[b, _LANES:_DIM] * mrows[b, _LANES:_DIM])
                acc = jnp.where(lanes == j, jnp.sum(s), acc)
            sl = pl.ds(gbase, _LANES)
            outv[sl] = acc + ubv[sl] + mbv[sl]

        pltpu.sync_copy(outv, out_hbm.at[pl.ds(base, _PER_WORKER)])

    return k


def kernel(user_ids, movie_ids, user_emb, movie_emb, user_bias, movie_bias):
    uids = user_ids.astype(jnp.int32).reshape(_NUM_WORKERS, _NUM_CHUNKS, _CHUNK)
    mids = movie_ids.astype(jnp.int32).reshape(_NUM_WORKERS, _NUM_CHUNKS, _CHUNK)
    ubias = user_bias.reshape(-1)
    mbias = movie_bias.reshape(-1)
    k = _make_kernel(user_emb.shape[0], movie_emb.shape[0])
    return k(uids, mids, user_emb, movie_emb, ubias, mbias)
